# Initial kernel scaffold; baseline (speedup 1.0000x reference)
import jax
import jax.numpy as jnp
from jax.experimental import pallas as pl


def kernel(x, edge_index, batch, graphs_mask, y, W, b, lin_W, lin_b):
    raise NotImplementedError("write your pallas kernel here")


# speedup vs baseline: 8.6357x; 8.6357x over previous
import functools

import jax
import jax.numpy as jnp
from jax import lax
from jax.experimental import pallas as pl
from jax.experimental.pallas import tpu as pltpu
from jax.experimental.pallas import tpu_sc as plsc

N_NODES = 10000
N_EDGES = 320000
D_FEAT = 128
HIDDEN = 128
N_CLASSES = 10
BATCH_SIZE = 1024

NC = 2
NS = 16
NW = NC * NS
CH = 128
NCHUNK = 80
EPT = CH * NCHUNK
E_PAD = EPT * NW
NPAD = 10240
ZRS = NPAD // NS

@functools.cache
def _sc_kernels():
    mesh = plsc.VectorSubcoreMesh(core_axis_name="c", subcore_axis_name="s")

    @functools.partial(
        pl.kernel,
        mesh=mesh,
        out_type=jax.ShapeDtypeStruct((NC, NPAD, D_FEAT), jnp.float32),
        scratch_types=[
            pltpu.VMEM((NCHUNK, CH), jnp.int32),
            pltpu.VMEM((NCHUNK, CH), jnp.int32),
            pltpu.VMEM((CH, D_FEAT), jnp.float32),
            pltpu.VMEM_SHARED((NPAD, D_FEAT), jnp.float32),
            pltpu.SemaphoreType.DMA,
        ],
    )
    def _edge_kernel(u_hbm, row_hbm, col_hbm, zeros_hbm, out_hbm,
                     rowbuf, colbuf, rows_v, aggsh, sem):
        cid = lax.axis_index("c")
        sid = lax.axis_index("s")
        wid = sid * NC + cid
        pltpu.sync_copy(row_hbm.at[wid], rowbuf)
        pltpu.sync_copy(col_hbm.at[wid], colbuf)
        pltpu.sync_copy(zeros_hbm.at[pl.ds(sid * ZRS, ZRS)],
                        aggsh.at[pl.ds(sid * ZRS, ZRS)])
        plsc.subcore_barrier()

        def body(i, carry):
            pltpu.async_copy(u_hbm.at[rowbuf.at[i]], rows_v, sem).wait()
            pltpu.sync_copy(rows_v, aggsh.at[colbuf.at[i]], add=True)
            return carry

        lax.fori_loop(0, NCHUNK, body, 0)
        plsc.subcore_barrier()
        pltpu.sync_copy(aggsh.at[pl.ds(sid * ZRS, ZRS)],
                        out_hbm.at[cid, pl.ds(sid * ZRS, ZRS)])

    return _edge_kernel


EB = 2048
NEB = E_PAD // EB


def _hist_body(col_ref, deg_ref):
    i = pl.program_id(0)

    @pl.when(i == 0)
    def _init():
        deg_ref[...] = jnp.zeros_like(deg_ref)

    c = col_ref[0, 0]
    hi = c >> 7
    lo = c & 127
    ind_hi = (hi[:, None] == lax.broadcasted_iota(jnp.int32, (EB, 128), 1)
              ).astype(jnp.float32)
    ind_lo = (lo[:, None] == lax.broadcasted_iota(jnp.int32, (EB, 128), 1)
              ).astype(jnp.float32)
    deg_ref[...] += lax.dot_general(
        ind_hi, ind_lo, (((0,), (0,)), ((), ())),
        preferred_element_type=jnp.float32)


def _deg_to_dis(deg):
    return jnp.where(deg > 0, lax.rsqrt(jnp.maximum(deg, 1e-12)), 0.0)


def _xw_body(x_ref, w_ref, deg_ref, u_ref):
    xw = jnp.dot(x_ref[...], w_ref[...], preferred_element_type=jnp.float32)
    u_ref[...] = xw * _deg_to_dis(deg_ref[...])


RB = 1000
NRB = N_NODES // RB


def _final_body(aggp_ref, deg_ref, batch_ref, b_ref, linw_ref, linb_ref,
                out_ref, pooled_acc, cnt_acc):
    i = pl.program_id(0)

    @pl.when(i == 0)
    def _init():
        pooled_acc[...] = jnp.zeros_like(pooled_acc)
        cnt_acc[...] = jnp.zeros_like(cnt_acc)

    a = aggp_ref[0] + aggp_ref[1]
    h = jnp.maximum(a * _deg_to_dis(deg_ref[...]) + b_ref[...], 0.0)
    bt = batch_ref[0, 0]
    ind = (bt[:, None] == lax.broadcasted_iota(jnp.int32, (RB, BATCH_SIZE), 1)
           ).astype(jnp.float32)
    pooled_acc[...] += lax.dot_general(
        ind, h, (((0,), (0,)), ((), ())), preferred_element_type=jnp.float32)
    cnt_acc[...] += jnp.sum(ind, axis=0)

    @pl.when(i == NRB - 1)
    def _fin():
        pooled = pooled_acc[...] / jnp.maximum(cnt_acc[...], 1.0)[:, None]
        out_ref[...] = jnp.dot(pooled, linw_ref[...],
                               preferred_element_type=jnp.float32) + linb_ref[...]


def kernel(x, edge_index, batch, graphs_mask, y, W, b, lin_W, lin_b):
    row = edge_index[0].astype(jnp.int32)
    col = edge_index[1].astype(jnp.int32)
    npad = E_PAD - N_EDGES
    row_p = jnp.concatenate([row, jnp.zeros((npad,), jnp.int32)])
    col_p = jnp.concatenate([col, jnp.full((npad,), N_NODES, jnp.int32)])
    row_r = row_p.reshape(NW, NCHUNK, CH)
    col_r = col_p.reshape(NW, NCHUNK, CH)
    zeros_nd = jnp.zeros((NPAD, D_FEAT), jnp.float32)
    batch_r = batch.astype(jnp.int32).reshape(NRB, 1, RB)

    _edge_kernel = _sc_kernels()

    col_hist = col_p.reshape(NEB, 1, EB)
    deg2d = pl.pallas_call(
        _hist_body,
        grid=(NEB,),
        in_specs=[pl.BlockSpec((1, 1, EB), lambda i: (i, 0, 0))],
        out_specs=pl.BlockSpec((128, 128), lambda i: (0, 0)),
        out_shape=jax.ShapeDtypeStruct((128, 128), jnp.float32),
    )(col_hist)
    deg2 = deg2d.reshape(-1)[:N_NODES].reshape(N_NODES, 1)

    u = pl.pallas_call(
        _xw_body,
        grid=(NRB,),
        in_specs=[
            pl.BlockSpec((RB, D_FEAT), lambda i: (i, 0)),
            pl.BlockSpec((D_FEAT, HIDDEN), lambda i: (0, 0)),
            pl.BlockSpec((RB, 1), lambda i: (i, 0)),
        ],
        out_specs=pl.BlockSpec((RB, HIDDEN), lambda i: (i, 0)),
        out_shape=jax.ShapeDtypeStruct((N_NODES, HIDDEN), jnp.float32),
    )(x, W, deg2)

    agg_parts = _edge_kernel(u, row_r, col_r, zeros_nd)

    linw_pad = jnp.zeros((HIDDEN, 128), jnp.float32).at[:, :N_CLASSES].set(lin_W)
    linb_pad = jnp.zeros((128,), jnp.float32).at[:N_CLASSES].set(lin_b)

    logits_pad = pl.pallas_call(
        _final_body,
        grid=(NRB,),
        in_specs=[
            pl.BlockSpec((NC, RB, HIDDEN), lambda i: (0, i, 0)),
            pl.BlockSpec((RB, 1), lambda i: (i, 0)),
            pl.BlockSpec((1, 1, RB), lambda i: (i, 0, 0)),
            pl.BlockSpec((HIDDEN,), lambda i: (0,)),
            pl.BlockSpec((HIDDEN, 128), lambda i: (0, 0)),
            pl.BlockSpec((128,), lambda i: (0,)),
        ],
        out_specs=pl.BlockSpec((BATCH_SIZE, 128), lambda i: (0, 0)),
        out_shape=jax.ShapeDtypeStruct((BATCH_SIZE, 128), jnp.float32),
        scratch_shapes=[
            pltpu.VMEM((BATCH_SIZE, HIDDEN), jnp.float32),
            pltpu.VMEM((BATCH_SIZE,), jnp.float32),
        ],
    )(agg_parts, deg2, batch_r, b, linw_pad, linb_pad)

    return logits_pad[:, :N_CLASSES]


# speedup vs baseline: 9.4866x; 1.0985x over previous
import functools

import jax
import jax.numpy as jnp
from jax import lax
from jax.experimental import pallas as pl
from jax.experimental.pallas import tpu as pltpu
from jax.experimental.pallas import tpu_sc as plsc

N_NODES = 10000
N_EDGES = 320000
D_FEAT = 128
HIDDEN = 128
N_CLASSES = 10
BATCH_SIZE = 1024

NC = 2
NS = 16
NW = NC * NS
CH = 128
NCHUNK = 80
EPT = CH * NCHUNK
E_PAD = EPT * NW
NPAD = 10240
ZRS = NPAD // NS

@functools.cache
def _sc_kernels():
    mesh = plsc.VectorSubcoreMesh(core_axis_name="c", subcore_axis_name="s")

    @functools.partial(
        pl.kernel,
        mesh=mesh,
        out_type=jax.ShapeDtypeStruct((NC, NPAD, D_FEAT), jnp.float32),
        scratch_types=[
            pltpu.VMEM((NCHUNK // 2, CH), jnp.int32),
            pltpu.VMEM((NCHUNK // 2, CH), jnp.int32),
            pltpu.VMEM((CH, D_FEAT), jnp.float32),
            pltpu.VMEM((CH, D_FEAT), jnp.float32),
            pltpu.VMEM_SHARED((NPAD, D_FEAT), jnp.float32),
            pltpu.SemaphoreType.DMA,
            pltpu.SemaphoreType.DMA,
        ],
    )
    def _edge_kernel(u_hbm, row_hbm, col_hbm, zeros_hbm, out_hbm,
                     rowbuf, colbuf, rows_a, rows_b, aggsh, sema, semb):
        cid = lax.axis_index("c")
        sid = lax.axis_index("s")
        wid = sid * NC + cid
        HC = NCHUNK // 2
        pltpu.sync_copy(zeros_hbm.at[pl.ds(sid * ZRS, ZRS)],
                        aggsh.at[pl.ds(sid * ZRS, ZRS)])
        plsc.subcore_barrier()

        for half in range(2):
            pltpu.sync_copy(row_hbm.at[wid, pl.ds(half * HC, HC)], rowbuf)
            pltpu.sync_copy(col_hbm.at[wid, pl.ds(half * HC, HC)], colbuf)
            pltpu.async_copy(u_hbm.at[rowbuf.at[0]], rows_a, sema)

            def body(g, carry):
                i0 = 2 * g
                pltpu.make_async_copy(u_hbm.at[rowbuf.at[i0]], rows_a,
                                      sema).wait()
                pltpu.async_copy(u_hbm.at[rowbuf.at[i0 + 1]], rows_b, semb)
                pltpu.sync_copy(rows_a, aggsh.at[colbuf.at[i0]], add=True)
                pltpu.make_async_copy(u_hbm.at[rowbuf.at[i0 + 1]], rows_b,
                                      semb).wait()

                @pl.when(g < HC // 2 - 1)
                def _next():
                    pltpu.async_copy(u_hbm.at[rowbuf.at[i0 + 2]], rows_a, sema)

                pltpu.sync_copy(rows_b, aggsh.at[colbuf.at[i0 + 1]], add=True)
                return carry

            lax.fori_loop(0, HC // 2, body, 0)
        plsc.subcore_barrier()
        pltpu.sync_copy(aggsh.at[pl.ds(sid * ZRS, ZRS)],
                        out_hbm.at[cid, pl.ds(sid * ZRS, ZRS)])

    return _edge_kernel


EB = 8192
NEB = E_PAD // EB


def _hist_body(col_ref, deg_ref):
    i = pl.program_id(0)

    @pl.when(i == 0)
    def _init():
        deg_ref[...] = jnp.zeros_like(deg_ref)

    c = col_ref[0, 0]
    hi = c >> 7
    lo = c & 127
    ind_hi = (hi[:, None] == lax.broadcasted_iota(jnp.int32, (EB, 128), 1)
              ).astype(jnp.bfloat16)
    ind_lo = (lo[:, None] == lax.broadcasted_iota(jnp.int32, (EB, 128), 1)
              ).astype(jnp.bfloat16)
    deg_ref[...] += lax.dot_general(
        ind_hi, ind_lo, (((0,), (0,)), ((), ())),
        preferred_element_type=jnp.float32)


def _deg_to_dis(deg):
    return jnp.where(deg > 0, lax.rsqrt(jnp.maximum(deg, 1e-12)), 0.0)


def _xw_body(x_ref, w_ref, deg_ref, u_ref):
    xw = jnp.dot(x_ref[...], w_ref[...], preferred_element_type=jnp.float32)
    u_ref[...] = xw * _deg_to_dis(deg_ref[...])


RB = 1000
NRB = N_NODES // RB


def _final_body(aggp_ref, deg_ref, batch_ref, b_ref, linw_ref, linb_ref,
                out_ref, pooled_acc, cnt_acc):
    i = pl.program_id(0)

    @pl.when(i == 0)
    def _init():
        pooled_acc[...] = jnp.zeros_like(pooled_acc)
        cnt_acc[...] = jnp.zeros_like(cnt_acc)

    a = aggp_ref[0] + aggp_ref[1]
    h = jnp.maximum(a * _deg_to_dis(deg_ref[...]) + b_ref[...], 0.0)
    bt = batch_ref[0, 0]
    ind = (bt[:, None] == lax.broadcasted_iota(jnp.int32, (RB, BATCH_SIZE), 1)
           ).astype(jnp.float32)
    pooled_acc[...] += lax.dot_general(
        ind, h, (((0,), (0,)), ((), ())), preferred_element_type=jnp.float32)
    cnt_acc[...] += jnp.sum(ind, axis=0)

    @pl.when(i == NRB - 1)
    def _fin():
        pooled = pooled_acc[...] / jnp.maximum(cnt_acc[...], 1.0)[:, None]
        out_ref[...] = jnp.dot(pooled, linw_ref[...],
                               preferred_element_type=jnp.float32) + linb_ref[...]


def kernel(x, edge_index, batch, graphs_mask, y, W, b, lin_W, lin_b):
    row = edge_index[0].astype(jnp.int32)
    col = edge_index[1].astype(jnp.int32)
    npad = E_PAD - N_EDGES
    row_p = jnp.concatenate([row, jnp.zeros((npad,), jnp.int32)])
    col_p = jnp.concatenate([col, jnp.full((npad,), N_NODES, jnp.int32)])
    row_r = row_p.reshape(NW, NCHUNK, CH)
    col_r = col_p.reshape(NW, NCHUNK, CH)
    zeros_nd = jnp.zeros((NPAD, D_FEAT), jnp.float32)
    batch_r = batch.astype(jnp.int32).reshape(NRB, 1, RB)

    _edge_kernel = _sc_kernels()

    col_hist = col_p.reshape(NEB, 1, EB)
    deg2d = pl.pallas_call(
        _hist_body,
        grid=(NEB,),
        in_specs=[pl.BlockSpec((1, 1, EB), lambda i: (i, 0, 0))],
        out_specs=pl.BlockSpec((128, 128), lambda i: (0, 0)),
        out_shape=jax.ShapeDtypeStruct((128, 128), jnp.float32),
    )(col_hist)
    deg2 = deg2d.reshape(-1)[:N_NODES].reshape(N_NODES, 1)

    u = pl.pallas_call(
        _xw_body,
        grid=(NRB,),
        in_specs=[
            pl.BlockSpec((RB, D_FEAT), lambda i: (i, 0)),
            pl.BlockSpec((D_FEAT, HIDDEN), lambda i: (0, 0)),
            pl.BlockSpec((RB, 1), lambda i: (i, 0)),
        ],
        out_specs=pl.BlockSpec((RB, HIDDEN), lambda i: (i, 0)),
        out_shape=jax.ShapeDtypeStruct((N_NODES, HIDDEN), jnp.float32),
    )(x, W, deg2)

    agg_parts = _edge_kernel(u, row_r, col_r, zeros_nd)

    linw_pad = jnp.zeros((HIDDEN, 128), jnp.float32).at[:, :N_CLASSES].set(lin_W)
    linb_pad = jnp.zeros((128,), jnp.float32).at[:N_CLASSES].set(lin_b)

    logits_pad = pl.pallas_call(
        _final_body,
        grid=(NRB,),
        in_specs=[
            pl.BlockSpec((NC, RB, HIDDEN), lambda i: (0, i, 0)),
            pl.BlockSpec((RB, 1), lambda i: (i, 0)),
            pl.BlockSpec((1, 1, RB), lambda i: (i, 0, 0)),
            pl.BlockSpec((HIDDEN,), lambda i: (0,)),
            pl.BlockSpec((HIDDEN, 128), lambda i: (0, 0)),
            pl.BlockSpec((128,), lambda i: (0,)),
        ],
        out_specs=pl.BlockSpec((BATCH_SIZE, 128), lambda i: (0, 0)),
        out_shape=jax.ShapeDtypeStruct((BATCH_SIZE, 128), jnp.float32),
        scratch_shapes=[
            pltpu.VMEM((BATCH_SIZE, HIDDEN), jnp.float32),
            pltpu.VMEM((BATCH_SIZE,), jnp.float32),
        ],
    )(agg_parts, deg2, batch_r, b, linw_pad, linb_pad)

    return logits_pad[:, :N_CLASSES]


# speedup vs baseline: 9.5143x; 1.0029x over previous
import functools

import jax
import jax.numpy as jnp
from jax import lax
from jax.experimental import pallas as pl
from jax.experimental.pallas import tpu as pltpu
from jax.experimental.pallas import tpu_sc as plsc

N_NODES = 10000
N_EDGES = 320000
D_FEAT = 128
HIDDEN = 128
N_CLASSES = 10
BATCH_SIZE = 1024

NC = 2
NS = 16
NW = NC * NS
CH = 128
NCHUNK = 80
EPT = CH * NCHUNK
E_PAD = EPT * NW
NPAD = 10240
ZRS = NPAD // NS

@functools.cache
def _sc_kernels():
    mesh = plsc.VectorSubcoreMesh(core_axis_name="c", subcore_axis_name="s")

    @functools.partial(
        pl.kernel,
        mesh=mesh,
        out_type=jax.ShapeDtypeStruct((NC, NPAD, D_FEAT), jnp.float32),
        scratch_types=[
            pltpu.VMEM((NCHUNK // 2, CH), jnp.int32),
            pltpu.VMEM((NCHUNK // 2, CH), jnp.int32),
            pltpu.VMEM((CH, D_FEAT), jnp.float32),
            pltpu.VMEM((CH, D_FEAT), jnp.float32),
            pltpu.VMEM_SHARED((NPAD, D_FEAT), jnp.float32),
            pltpu.SemaphoreType.DMA,
            pltpu.SemaphoreType.DMA,
        ],
    )
    def _edge_kernel(u_hbm, row_hbm, col_hbm, zeros_hbm, out_hbm,
                     rowbuf, colbuf, rows_a, rows_b, aggsh, sema, semb):
        cid = lax.axis_index("c")
        sid = lax.axis_index("s")
        wid = sid * NC + cid
        HC = NCHUNK // 2
        pltpu.sync_copy(zeros_hbm, rows_a)
        for k in range(ZRS // CH):
            pltpu.sync_copy(rows_a, aggsh.at[pl.ds(sid * ZRS + k * CH, CH)])
        plsc.subcore_barrier()

        for half in range(2):
            pltpu.sync_copy(row_hbm.at[wid, pl.ds(half * HC, HC)], rowbuf)
            pltpu.sync_copy(col_hbm.at[wid, pl.ds(half * HC, HC)], colbuf)
            pltpu.async_copy(u_hbm.at[rowbuf.at[0]], rows_a, sema)

            def body(g, carry):
                i0 = 2 * g
                pltpu.make_async_copy(u_hbm.at[rowbuf.at[i0]], rows_a,
                                      sema).wait()
                pltpu.async_copy(u_hbm.at[rowbuf.at[i0 + 1]], rows_b, semb)
                pltpu.sync_copy(rows_a, aggsh.at[colbuf.at[i0]], add=True)
                pltpu.make_async_copy(u_hbm.at[rowbuf.at[i0 + 1]], rows_b,
                                      semb).wait()

                @pl.when(g < HC // 2 - 1)
                def _next():
                    pltpu.async_copy(u_hbm.at[rowbuf.at[i0 + 2]], rows_a, sema)

                pltpu.sync_copy(rows_b, aggsh.at[colbuf.at[i0 + 1]], add=True)
                return carry

            lax.fori_loop(0, HC // 2, body, 0)
        plsc.subcore_barrier()
        bufs = (rows_a, rows_b)
        sems = (sema, semb)
        nko = ZRS // CH
        for k in range(nko):
            bk = k % 2
            if k >= 2:
                pltpu.make_async_copy(
                    bufs[bk],
                    out_hbm.at[cid, pl.ds(sid * ZRS + (k - 2) * CH, CH)],
                    sems[bk]).wait()
            pltpu.sync_copy(aggsh.at[pl.ds(sid * ZRS + k * CH, CH)], bufs[bk])
            pltpu.async_copy(bufs[bk],
                             out_hbm.at[cid, pl.ds(sid * ZRS + k * CH, CH)],
                             sems[bk])
        for k in range(max(0, nko - 2), nko):
            bk = k % 2
            pltpu.make_async_copy(
                bufs[bk], out_hbm.at[cid, pl.ds(sid * ZRS + k * CH, CH)],
                sems[bk]).wait()

    return _edge_kernel


EB = 8192
NEB = E_PAD // EB


def _hist_body(col_ref, deg_ref):
    i = pl.program_id(0)

    @pl.when(i == 0)
    def _init():
        deg_ref[...] = jnp.zeros_like(deg_ref)

    c = col_ref[0, 0]
    hi = c >> 7
    lo = c & 127
    ind_hi = (hi[:, None] == lax.broadcasted_iota(jnp.int32, (EB, 128), 1)
              ).astype(jnp.bfloat16)
    ind_lo = (lo[:, None] == lax.broadcasted_iota(jnp.int32, (EB, 128), 1)
              ).astype(jnp.bfloat16)
    deg_ref[...] += lax.dot_general(
        ind_hi, ind_lo, (((0,), (0,)), ((), ())),
        preferred_element_type=jnp.float32)


def _deg_to_dis(deg):
    return jnp.where(deg > 0, lax.rsqrt(jnp.maximum(deg, 1e-12)), 0.0)


def _xw_body(x_ref, w_ref, deg_ref, u_ref):
    xw = jnp.dot(x_ref[...], w_ref[...], preferred_element_type=jnp.float32)
    u_ref[...] = xw * _deg_to_dis(deg_ref[...])


RB = 1000
NRB = N_NODES // RB


def _final_body(aggp_ref, deg_ref, batch_ref, b_ref, linw_ref, linb_ref,
                out_ref, pooled_acc, cnt_acc):
    i = pl.program_id(0)

    @pl.when(i == 0)
    def _init():
        pooled_acc[...] = jnp.zeros_like(pooled_acc)
        cnt_acc[...] = jnp.zeros_like(cnt_acc)

    a = aggp_ref[0] + aggp_ref[1]
    h = jnp.maximum(a * _deg_to_dis(deg_ref[...]) + b_ref[...], 0.0)
    bt = batch_ref[0, 0]
    ind = (bt[:, None] == lax.broadcasted_iota(jnp.int32, (RB, BATCH_SIZE), 1)
           ).astype(jnp.float32)
    pooled_acc[...] += lax.dot_general(
        ind, h, (((0,), (0,)), ((), ())), preferred_element_type=jnp.float32)
    cnt_acc[...] += jnp.sum(ind, axis=0)

    @pl.when(i == NRB - 1)
    def _fin():
        pooled = pooled_acc[...] / jnp.maximum(cnt_acc[...], 1.0)[:, None]
        out_ref[...] = jnp.dot(pooled, linw_ref[...],
                               preferred_element_type=jnp.float32) + linb_ref[...]


def kernel(x, edge_index, batch, graphs_mask, y, W, b, lin_W, lin_b):
    row = edge_index[0].astype(jnp.int32)
    col = edge_index[1].astype(jnp.int32)
    npad = E_PAD - N_EDGES
    row_p = jnp.concatenate([row, jnp.zeros((npad,), jnp.int32)])
    col_p = jnp.concatenate([col, jnp.full((npad,), N_NODES, jnp.int32)])
    row_r = row_p.reshape(NW, NCHUNK, CH)
    col_r = col_p.reshape(NW, NCHUNK, CH)
    zeros_nd = jnp.zeros((CH, D_FEAT), jnp.float32)
    batch_r = batch.astype(jnp.int32).reshape(NRB, 1, RB)

    _edge_kernel = _sc_kernels()

    col_hist = col_p.reshape(NEB, 1, EB)
    deg2d = pl.pallas_call(
        _hist_body,
        grid=(NEB,),
        in_specs=[pl.BlockSpec((1, 1, EB), lambda i: (i, 0, 0))],
        out_specs=pl.BlockSpec((128, 128), lambda i: (0, 0)),
        out_shape=jax.ShapeDtypeStruct((128, 128), jnp.float32),
    )(col_hist)
    deg2 = deg2d.reshape(-1)[:N_NODES].reshape(N_NODES, 1)

    u = pl.pallas_call(
        _xw_body,
        grid=(NRB,),
        in_specs=[
            pl.BlockSpec((RB, D_FEAT), lambda i: (i, 0)),
            pl.BlockSpec((D_FEAT, HIDDEN), lambda i: (0, 0)),
            pl.BlockSpec((RB, 1), lambda i: (i, 0)),
        ],
        out_specs=pl.BlockSpec((RB, HIDDEN), lambda i: (i, 0)),
        out_shape=jax.ShapeDtypeStruct((N_NODES, HIDDEN), jnp.float32),
    )(x, W, deg2)

    agg_parts = _edge_kernel(u, row_r, col_r, zeros_nd)

    linw_pad = jnp.zeros((HIDDEN, 128), jnp.float32).at[:, :N_CLASSES].set(lin_W)
    linb_pad = jnp.zeros((128,), jnp.float32).at[:N_CLASSES].set(lin_b)

    logits_pad = pl.pallas_call(
        _final_body,
        grid=(NRB,),
        in_specs=[
            pl.BlockSpec((NC, RB, HIDDEN), lambda i: (0, i, 0)),
            pl.BlockSpec((RB, 1), lambda i: (i, 0)),
            pl.BlockSpec((1, 1, RB), lambda i: (i, 0, 0)),
            pl.BlockSpec((HIDDEN,), lambda i: (0,)),
            pl.BlockSpec((HIDDEN, 128), lambda i: (0, 0)),
            pl.BlockSpec((128,), lambda i: (0,)),
        ],
        out_specs=pl.BlockSpec((BATCH_SIZE, 128), lambda i: (0, 0)),
        out_shape=jax.ShapeDtypeStruct((BATCH_SIZE, 128), jnp.float32),
        scratch_shapes=[
            pltpu.VMEM((BATCH_SIZE, HIDDEN), jnp.float32),
            pltpu.VMEM((BATCH_SIZE,), jnp.float32),
        ],
    )(agg_parts, deg2, batch_r, b, linw_pad, linb_pad)

    return logits_pad[:, :N_CLASSES]


# speedup vs baseline: 9.5303x; 1.0017x over previous
import functools

import jax
import jax.numpy as jnp
from jax import lax
from jax.experimental import pallas as pl
from jax.experimental.pallas import tpu as pltpu
from jax.experimental.pallas import tpu_sc as plsc

N_NODES = 10000
N_EDGES = 320000
D_FEAT = 128
HIDDEN = 128
N_CLASSES = 10
BATCH_SIZE = 1024

NC = 2
NS = 16
NW = NC * NS
CH = 128
NCHUNK = 80
EPT = CH * NCHUNK
E_PAD = EPT * NW
NPAD = 10240
ZRS = NPAD // NS

@functools.cache
def _sc_kernels():
    mesh = plsc.VectorSubcoreMesh(core_axis_name="c", subcore_axis_name="s")

    @functools.partial(
        pl.kernel,
        mesh=mesh,
        out_type=jax.ShapeDtypeStruct((NC, NPAD, D_FEAT), jnp.float32),
        scratch_types=[
            pltpu.VMEM((NCHUNK // 2, CH), jnp.int32),
            pltpu.VMEM((NCHUNK // 2, CH), jnp.int32),
            pltpu.VMEM((CH, D_FEAT), jnp.float32),
            pltpu.VMEM((CH, D_FEAT), jnp.float32),
            pltpu.VMEM_SHARED((NPAD, D_FEAT), jnp.float32),
            pltpu.SemaphoreType.DMA,
            pltpu.SemaphoreType.DMA,
        ],
    )
    def _edge_kernel(u_hbm, row_hbm, col_hbm, zeros_hbm, out_hbm,
                     rowbuf, colbuf, rows_a, rows_b, aggsh, sema, semb):
        cid = lax.axis_index("c")
        sid = lax.axis_index("s")
        wid = sid * NC + cid
        HC = NCHUNK // 2
        with jax.named_scope("zinit"):
            pltpu.sync_copy(zeros_hbm, rows_a)
            for k in range(ZRS // CH):
                pltpu.sync_copy(rows_a, aggsh.at[pl.ds(sid * ZRS + k * CH, CH)])
            plsc.subcore_barrier()

        for half in range(2):
          with jax.named_scope(f"mainloop{half}"):
            pltpu.sync_copy(row_hbm.at[wid, pl.ds(half * HC, HC)], rowbuf)
            pltpu.sync_copy(col_hbm.at[wid, pl.ds(half * HC, HC)], colbuf)
            pltpu.async_copy(u_hbm.at[rowbuf.at[0]], rows_a, sema)

            def body(g, carry):
                i0 = 2 * g
                pltpu.make_async_copy(u_hbm.at[rowbuf.at[i0]], rows_a,
                                      sema).wait()
                pltpu.async_copy(u_hbm.at[rowbuf.at[i0 + 1]], rows_b, semb)
                pltpu.sync_copy(rows_a, aggsh.at[colbuf.at[i0]], add=True)
                pltpu.make_async_copy(u_hbm.at[rowbuf.at[i0 + 1]], rows_b,
                                      semb).wait()

                @pl.when(g < HC // 2 - 1)
                def _next():
                    pltpu.async_copy(u_hbm.at[rowbuf.at[i0 + 2]], rows_a, sema)

                pltpu.sync_copy(rows_b, aggsh.at[colbuf.at[i0 + 1]], add=True)
                return carry

            lax.fori_loop(0, HC // 2, body, 0)
        plsc.subcore_barrier()
        with jax.named_scope("copyout"):
            bufs = (rows_a, rows_b)
            sems = (sema, semb)
            nko = ZRS // CH
            for k in range(nko):
                bk = k % 2
                if k >= 2:
                    pltpu.make_async_copy(
                        bufs[bk],
                        out_hbm.at[cid, pl.ds(sid * ZRS + (k - 2) * CH, CH)],
                        sems[bk]).wait()
                pltpu.sync_copy(aggsh.at[pl.ds(sid * ZRS + k * CH, CH)],
                                bufs[bk])
                pltpu.async_copy(bufs[bk],
                                 out_hbm.at[cid, pl.ds(sid * ZRS + k * CH, CH)],
                                 sems[bk])
            for k in range(max(0, nko - 2), nko):
                bk = k % 2
                pltpu.make_async_copy(
                    bufs[bk], out_hbm.at[cid, pl.ds(sid * ZRS + k * CH, CH)],
                    sems[bk]).wait()

    return _edge_kernel


EB = 8192
NEB = E_PAD // EB


def _hist_body(col_ref, deg_ref):
    i = pl.program_id(0)

    @pl.when(i == 0)
    def _init():
        deg_ref[...] = jnp.zeros_like(deg_ref)

    c = col_ref[0, 0]
    hi = c >> 7
    lo = c & 127
    ind_hi = (hi[:, None] == lax.broadcasted_iota(jnp.int32, (EB, 128), 1)
              ).astype(jnp.bfloat16)
    ind_lo = (lo[:, None] == lax.broadcasted_iota(jnp.int32, (EB, 128), 1)
              ).astype(jnp.bfloat16)
    deg_ref[...] += lax.dot_general(
        ind_hi, ind_lo, (((0,), (0,)), ((), ())),
        preferred_element_type=jnp.float32)


def _deg_to_dis(deg):
    return jnp.where(deg > 0, lax.rsqrt(jnp.maximum(deg, 1e-12)), 0.0)


def _xw_body(x_ref, w_ref, deg_ref, u_ref):
    xw = jnp.dot(x_ref[...], w_ref[...], preferred_element_type=jnp.float32)
    u_ref[...] = xw * _deg_to_dis(deg_ref[...])


RB = 1000
NRB = N_NODES // RB


def _final_body(aggp_ref, deg_ref, batch_ref, b_ref, linw_ref, linb_ref,
                out_ref, pooled_acc, cnt_acc):
    i = pl.program_id(0)

    @pl.when(i == 0)
    def _init():
        pooled_acc[...] = jnp.zeros_like(pooled_acc)
        cnt_acc[...] = jnp.zeros_like(cnt_acc)

    a = aggp_ref[0] + aggp_ref[1]
    h = jnp.maximum(a * _deg_to_dis(deg_ref[...]) + b_ref[...], 0.0)
    bt = batch_ref[0, 0]
    ind = (bt[:, None] == lax.broadcasted_iota(jnp.int32, (RB, BATCH_SIZE), 1)
           ).astype(jnp.float32)
    pooled_acc[...] += lax.dot_general(
        ind, h, (((0,), (0,)), ((), ())), preferred_element_type=jnp.float32)
    cnt_acc[...] += jnp.sum(ind, axis=0)

    @pl.when(i == NRB - 1)
    def _fin():
        pooled = pooled_acc[...] / jnp.maximum(cnt_acc[...], 1.0)[:, None]
        out_ref[...] = jnp.dot(pooled, linw_ref[...],
                               preferred_element_type=jnp.float32) + linb_ref[...]


def kernel(x, edge_index, batch, graphs_mask, y, W, b, lin_W, lin_b):
    row = edge_index[0].astype(jnp.int32)
    col = edge_index[1].astype(jnp.int32)
    npad = E_PAD - N_EDGES
    row_p = jnp.concatenate([row, jnp.zeros((npad,), jnp.int32)])
    col_p = jnp.concatenate([col, jnp.full((npad,), N_NODES, jnp.int32)])
    row_r = row_p.reshape(NW, NCHUNK, CH)
    col_r = col_p.reshape(NW, NCHUNK, CH)
    zeros_nd = jnp.zeros((CH, D_FEAT), jnp.float32)
    batch_r = batch.astype(jnp.int32).reshape(NRB, 1, RB)

    _edge_kernel = _sc_kernels()

    col_hist = col_p.reshape(NEB, 1, EB)
    deg2d = pl.pallas_call(
        _hist_body,
        grid=(NEB,),
        in_specs=[pl.BlockSpec((1, 1, EB), lambda i: (i, 0, 0))],
        out_specs=pl.BlockSpec((128, 128), lambda i: (0, 0)),
        out_shape=jax.ShapeDtypeStruct((128, 128), jnp.float32),
    )(col_hist)
    deg2 = deg2d.reshape(-1)[:N_NODES].reshape(N_NODES, 1)

    u = pl.pallas_call(
        _xw_body,
        grid=(NRB,),
        in_specs=[
            pl.BlockSpec((RB, D_FEAT), lambda i: (i, 0)),
            pl.BlockSpec((D_FEAT, HIDDEN), lambda i: (0, 0)),
            pl.BlockSpec((RB, 1), lambda i: (i, 0)),
        ],
        out_specs=pl.BlockSpec((RB, HIDDEN), lambda i: (i, 0)),
        out_shape=jax.ShapeDtypeStruct((N_NODES, HIDDEN), jnp.float32),
    )(x, W, deg2)

    agg_parts = _edge_kernel(u, row_r, col_r, zeros_nd)

    linw_pad = jnp.zeros((HIDDEN, 128), jnp.float32).at[:, :N_CLASSES].set(lin_W)
    linb_pad = jnp.zeros((128,), jnp.float32).at[:N_CLASSES].set(lin_b)

    logits_pad = pl.pallas_call(
        _final_body,
        grid=(NRB,),
        in_specs=[
            pl.BlockSpec((NC, RB, HIDDEN), lambda i: (0, i, 0)),
            pl.BlockSpec((RB, 1), lambda i: (i, 0)),
            pl.BlockSpec((1, 1, RB), lambda i: (i, 0, 0)),
            pl.BlockSpec((HIDDEN,), lambda i: (0,)),
            pl.BlockSpec((HIDDEN, 128), lambda i: (0, 0)),
            pl.BlockSpec((128,), lambda i: (0,)),
        ],
        out_specs=pl.BlockSpec((BATCH_SIZE, 128), lambda i: (0, 0)),
        out_shape=jax.ShapeDtypeStruct((BATCH_SIZE, 128), jnp.float32),
        scratch_shapes=[
            pltpu.VMEM((BATCH_SIZE, HIDDEN), jnp.float32),
            pltpu.VMEM((BATCH_SIZE,), jnp.float32),
        ],
    )(agg_parts, deg2, batch_r, b, linw_pad, linb_pad)

    return logits_pad[:, :N_CLASSES]


# speedup vs baseline: 9.6871x; 1.0165x over previous
import functools

import jax
import jax.numpy as jnp
from jax import lax
from jax.experimental import pallas as pl
from jax.experimental.pallas import tpu as pltpu
from jax.experimental.pallas import tpu_sc as plsc

N_NODES = 10000
N_EDGES = 320000
D_FEAT = 128
HIDDEN = 128
N_CLASSES = 10
BATCH_SIZE = 1024

NC = 2
NS = 16
NW = NC * NS
CH = 128
NCHUNK = 80
NSTAGE = 2
NBUF = 2
EPT = CH * NCHUNK
E_PAD = EPT * NW
NPAD = 10240
ZRS = NPAD // NS

@functools.cache
def _sc_kernels():
    mesh = plsc.VectorSubcoreMesh(core_axis_name="c", subcore_axis_name="s")

    @functools.partial(
        pl.kernel,
        mesh=mesh,
        out_type=jax.ShapeDtypeStruct((NC, NPAD, D_FEAT), jnp.float32),
        scratch_types=[
            pltpu.VMEM((NCHUNK // NSTAGE, CH), jnp.int32),
            pltpu.VMEM((NCHUNK // NSTAGE, CH), jnp.int32),
        ] + [pltpu.VMEM((CH, D_FEAT), jnp.float32) for _ in range(NBUF)] + [
            pltpu.VMEM_SHARED((NPAD, D_FEAT), jnp.float32),
        ] + [pltpu.SemaphoreType.DMA for _ in range(2 * NBUF)],
    )
    def _edge_kernel(u_hbm, row_hbm, col_hbm, zeros_hbm, out_hbm,
                     rowbuf, colbuf, *rest):
        bufs = rest[:NBUF]
        aggsh = rest[NBUF]
        sems = rest[NBUF + 1:]
        cid = lax.axis_index("c")
        sid = lax.axis_index("s")
        wid = sid * NC + cid
        HC = NCHUNK // NSTAGE

        def gather2(i, b):
            for h in range(2):
                pltpu.async_copy(
                    u_hbm.at[rowbuf.at[i, pl.ds(h * 64, 64)]],
                    bufs[b].at[pl.ds(h * 64, 64)], sems[2 * b + h])

        def gather2_wait(i, b):
            for h in range(2):
                pltpu.make_async_copy(
                    u_hbm.at[rowbuf.at[i, pl.ds(h * 64, 64)]],
                    bufs[b].at[pl.ds(h * 64, 64)], sems[2 * b + h]).wait()
        with jax.named_scope("zinit"):
            pltpu.sync_copy(zeros_hbm, bufs[0])
            for k in range(ZRS // CH):
                pltpu.sync_copy(bufs[0],
                                aggsh.at[pl.ds(sid * ZRS + k * CH, CH)])
            plsc.subcore_barrier()

        for stage in range(NSTAGE):
          with jax.named_scope(f"mainloop{stage}"):
            pltpu.sync_copy(row_hbm.at[wid, pl.ds(stage * HC, HC)], rowbuf)
            pltpu.sync_copy(col_hbm.at[wid, pl.ds(stage * HC, HC)], colbuf)
            for b in range(NBUF):
                gather2(b, b)

            def body(q, carry):
                for b in range(NBUF):
                    i = q * NBUF + b
                    gather2_wait(i, b)
                    pltpu.sync_copy(bufs[b], aggsh.at[colbuf.at[i]], add=True)

                    @pl.when(q < HC // NBUF - 1)
                    def _pf():
                        gather2(i + NBUF, b)
                return carry

            lax.fori_loop(0, HC // NBUF, body, 0)
        plsc.subcore_barrier()
        with jax.named_scope("copyout"):
            nko = ZRS // CH
            for k in range(nko):
                bk = k % 2
                if k >= 2:
                    pltpu.make_async_copy(
                        bufs[bk],
                        out_hbm.at[cid, pl.ds(sid * ZRS + (k - 2) * CH, CH)],
                        sems[bk]).wait()
                pltpu.sync_copy(aggsh.at[pl.ds(sid * ZRS + k * CH, CH)],
                                bufs[bk])
                pltpu.async_copy(bufs[bk],
                                 out_hbm.at[cid, pl.ds(sid * ZRS + k * CH, CH)],
                                 sems[bk])
            for k in range(max(0, nko - 2), nko):
                bk = k % 2
                pltpu.make_async_copy(
                    bufs[bk], out_hbm.at[cid, pl.ds(sid * ZRS + k * CH, CH)],
                    sems[bk]).wait()

    return _edge_kernel


EB = 8192
NEB = E_PAD // EB


def _hist_body(col_ref, deg_ref):
    i = pl.program_id(0)

    @pl.when(i == 0)
    def _init():
        deg_ref[...] = jnp.zeros_like(deg_ref)

    c = col_ref[0, 0]
    hi = c >> 7
    lo = c & 127
    ind_hi = (hi[:, None] == lax.broadcasted_iota(jnp.int32, (EB, 128), 1)
              ).astype(jnp.bfloat16)
    ind_lo = (lo[:, None] == lax.broadcasted_iota(jnp.int32, (EB, 128), 1)
              ).astype(jnp.bfloat16)
    deg_ref[...] += lax.dot_general(
        ind_hi, ind_lo, (((0,), (0,)), ((), ())),
        preferred_element_type=jnp.float32)


def _deg_to_dis(deg):
    return jnp.where(deg > 0, lax.rsqrt(jnp.maximum(deg, 1e-12)), 0.0)


def _xw_body(x_ref, w_ref, deg_ref, u_ref):
    xw = jnp.dot(x_ref[...], w_ref[...], preferred_element_type=jnp.float32)
    u_ref[...] = xw * _deg_to_dis(deg_ref[...])


RB = 1000
NRB = N_NODES // RB


def _final_body(aggp_ref, deg_ref, batch_ref, b_ref, linw_ref, linb_ref,
                out_ref, pooled_acc, cnt_acc):
    i = pl.program_id(0)

    @pl.when(i == 0)
    def _init():
        pooled_acc[...] = jnp.zeros_like(pooled_acc)
        cnt_acc[...] = jnp.zeros_like(cnt_acc)

    a = aggp_ref[0] + aggp_ref[1]
    h = jnp.maximum(a * _deg_to_dis(deg_ref[...]) + b_ref[...], 0.0)
    bt = batch_ref[0, 0]
    ind = (bt[:, None] == lax.broadcasted_iota(jnp.int32, (RB, BATCH_SIZE), 1)
           ).astype(jnp.float32)
    pooled_acc[...] += lax.dot_general(
        ind, h, (((0,), (0,)), ((), ())), preferred_element_type=jnp.float32)
    cnt_acc[...] += jnp.sum(ind, axis=0)

    @pl.when(i == NRB - 1)
    def _fin():
        pooled = pooled_acc[...] / jnp.maximum(cnt_acc[...], 1.0)[:, None]
        out_ref[...] = jnp.dot(pooled, linw_ref[...],
                               preferred_element_type=jnp.float32) + linb_ref[...]


def kernel(x, edge_index, batch, graphs_mask, y, W, b, lin_W, lin_b):
    row = edge_index[0].astype(jnp.int32)
    col = edge_index[1].astype(jnp.int32)
    npad = E_PAD - N_EDGES
    row_p = jnp.concatenate([row, jnp.zeros((npad,), jnp.int32)])
    col_p = jnp.concatenate([col, jnp.full((npad,), N_NODES, jnp.int32)])
    row_r = row_p.reshape(NW, NCHUNK, CH)
    col_r = col_p.reshape(NW, NCHUNK, CH)
    zeros_nd = jnp.zeros((CH, D_FEAT), jnp.float32)
    batch_r = batch.astype(jnp.int32).reshape(NRB, 1, RB)

    _edge_kernel = _sc_kernels()

    col_hist = col_p.reshape(NEB, 1, EB)
    deg2d = pl.pallas_call(
        _hist_body,
        grid=(NEB,),
        in_specs=[pl.BlockSpec((1, 1, EB), lambda i: (i, 0, 0))],
        out_specs=pl.BlockSpec((128, 128), lambda i: (0, 0)),
        out_shape=jax.ShapeDtypeStruct((128, 128), jnp.float32),
    )(col_hist)
    deg2 = deg2d.reshape(-1)[:N_NODES].reshape(N_NODES, 1)

    u = pl.pallas_call(
        _xw_body,
        grid=(NRB,),
        in_specs=[
            pl.BlockSpec((RB, D_FEAT), lambda i: (i, 0)),
            pl.BlockSpec((D_FEAT, HIDDEN), lambda i: (0, 0)),
            pl.BlockSpec((RB, 1), lambda i: (i, 0)),
        ],
        out_specs=pl.BlockSpec((RB, HIDDEN), lambda i: (i, 0)),
        out_shape=jax.ShapeDtypeStruct((N_NODES, HIDDEN), jnp.float32),
    )(x, W, deg2)

    agg_parts = _edge_kernel(u, row_r, col_r, zeros_nd)

    linw_pad = jnp.zeros((HIDDEN, 128), jnp.float32).at[:, :N_CLASSES].set(lin_W)
    linb_pad = jnp.zeros((128,), jnp.float32).at[:N_CLASSES].set(lin_b)

    logits_pad = pl.pallas_call(
        _final_body,
        grid=(NRB,),
        in_specs=[
            pl.BlockSpec((NC, RB, HIDDEN), lambda i: (0, i, 0)),
            pl.BlockSpec((RB, 1), lambda i: (i, 0)),
            pl.BlockSpec((1, 1, RB), lambda i: (i, 0, 0)),
            pl.BlockSpec((HIDDEN,), lambda i: (0,)),
            pl.BlockSpec((HIDDEN, 128), lambda i: (0, 0)),
            pl.BlockSpec((128,), lambda i: (0,)),
        ],
        out_specs=pl.BlockSpec((BATCH_SIZE, 128), lambda i: (0, 0)),
        out_shape=jax.ShapeDtypeStruct((BATCH_SIZE, 128), jnp.float32),
        scratch_shapes=[
            pltpu.VMEM((BATCH_SIZE, HIDDEN), jnp.float32),
            pltpu.VMEM((BATCH_SIZE,), jnp.float32),
        ],
    )(agg_parts, deg2, batch_r, b, linw_pad, linb_pad)

    return logits_pad[:, :N_CLASSES]
